# baseline (speedup 1.0000x reference)
import functools

import jax
import jax.numpy as jnp
from jax import lax
from jax.experimental import pallas as pl
from jax.experimental.pallas import tpu as pltpu
from jax.experimental.pallas import tpu_sc as plsc

N = 10000
E = 320000
H = 128
DEPTH = 3
M = 512
BN_EPS = 1e-5

NC = 2
NS = 16
NW = NC * NS
C = 56
TCH = 180
EW = C * TCH
E_PAD = EW * NW
NZ = 16
N_ACC = 10112
RPT = N_ACC // NS
NBUF = 6


def _sc_agg_body(h_hbm, e_hbm, zeros_hbm, out_hbm,
                 acc_sh, idx0, isem0, rows0, gsem0, ssem0,
                 idx1, isem1, rows1, gsem1, ssem1,
                 idx2, isem2, rows2, gsem2, ssem2,
                 idx3, isem3, rows3, gsem3, ssem3,
                 idx4, isem4, rows4, gsem4, ssem4,
                 idx5, isem5, rows5, gsem5, ssem5):
    cid = lax.axis_index("c")
    sid = lax.axis_index("s")
    wid = sid * NC + cid

    pltpu.sync_copy(zeros_hbm.at[pl.ds(sid * RPT, RPT)],
                    acc_sh.at[pl.ds(sid * RPT, RPT)])
    plsc.subcore_barrier()

    idx = (idx0, idx1, idx2, idx3, idx4, idx5)
    isem = (isem0, isem1, isem2, isem3, isem4, isem5)
    rows = (rows0, rows1, rows2, rows3, rows4, rows5)
    gsem = (gsem0, gsem1, gsem2, gsem3, gsem4, gsem5)
    ssem = (ssem0, ssem1, ssem2, ssem3, ssem4, ssem5)

    def issue_idx(c, b):
        pltpu.async_copy(e_hbm.at[wid, c], idx[b], isem[b])

    def wait_idx(c, b):
        pltpu.make_async_copy(e_hbm.at[wid, c], idx[b], isem[b]).wait()

    def issue_gather(c, b):
        pltpu.async_copy(h_hbm.at[idx[b].at[0]], rows[b], gsem[b])

    def wait_gather(c, b):
        pltpu.make_async_copy(h_hbm.at[idx[b].at[0]], rows[b], gsem[b]).wait()

    def issue_scatter(c, b):
        pltpu.async_copy(rows[b], acc_sh.at[idx[b].at[1]], ssem[b], add=True)

    def wait_scatter(c, b):
        pltpu.make_async_copy(rows[b], acc_sh.at[idx[b].at[1]],
                              ssem[b]).wait()

    issue_idx(0, 0)
    issue_idx(1, 1)
    issue_idx(2, 2)
    wait_idx(0, 0)
    issue_gather(0, 0)
    wait_idx(1, 1)
    issue_gather(1, 1)
    for j in range(3):
        issue_idx(j + 3, j + 3)
        wait_idx(j + 2, j + 2)
        issue_gather(j + 2, j + 2)
        wait_gather(j, j)
        issue_scatter(j, j)

    @pl.loop(0, (TCH - 6) // NBUF)
    def _ring(k):
        for t in range(NBUF):
            j = 3 + k * NBUF + t
            b0 = (3 + t) % NBUF
            b2 = (5 + t) % NBUF
            b3 = t
            wait_scatter(j - 3, b3)
            issue_idx(j + 3, b3)
            wait_idx(j + 2, b2)
            issue_gather(j + 2, b2)
            wait_gather(j, b0)
            issue_scatter(j, b0)

    wait_idx(TCH - 1, (TCH - 1) % NBUF)
    issue_gather(TCH - 1, (TCH - 1) % NBUF)
    for j in range(TCH - 3, TCH):
        wait_gather(j, j % NBUF)
        issue_scatter(j, j % NBUF)
    for b in range(NBUF):
        wait_scatter(0, b)

    plsc.subcore_barrier()

    pltpu.sync_copy(acc_sh.at[pl.ds(sid * RPT, RPT)],
                    out_hbm.at[cid, pl.ds(sid * RPT, RPT)])


@jax.jit
def _sc_agg(h_tab, edges, zeros):
    mesh = plsc.VectorSubcoreMesh(core_axis_name="c", subcore_axis_name="s",
                                  num_cores=NC, num_subcores=NS)
    return pl.kernel(
        _sc_agg_body,
        out_type=jax.ShapeDtypeStruct((NC, N_ACC, H), jnp.float32),
        mesh=mesh,
        scratch_types=[
            pltpu.VMEM_SHARED((N_ACC, H), jnp.float32),
        ] + [
            s for _ in range(NBUF) for s in (
                pltpu.VMEM((2, C), jnp.int32),
                pltpu.SemaphoreType.DMA,
                pltpu.VMEM((C, H), jnp.float32),
                pltpu.SemaphoreType.DMA,
                pltpu.SemaphoreType.DMA,
            )
        ],
    )(h_tab, edges, zeros)


def _matmul_t(a, w):
    return lax.dot_general(a, w, (((1,), (1,)), ((), ())),
                           preferred_element_type=jnp.float32)


def _bn(t, g, b):
    mu = jnp.mean(t, axis=0, keepdims=True)
    d = t - mu
    var = jnp.mean(d * d, axis=0, keepdims=True)
    return g * (d * lax.rsqrt(var + BN_EPS)) + b


def _store_tab(o_ref, t):
    o_ref[:N, :] = t
    o_ref[N:, :] = jnp.zeros((NZ, H), jnp.float32)


def _proj_body(x_ref, w_ref, b_ref, o_ref):
    _store_tab(o_ref, jnp.maximum(
        _matmul_t(x_ref[...], w_ref[...]) + b_ref[...], 0.0))


@jax.jit
def _tc_proj(x, w_in, b_in):
    return pl.pallas_call(
        _proj_body,
        out_shape=jax.ShapeDtypeStruct((N + NZ, H), jnp.float32),
    )(x, w_in, b_in.reshape(1, H))


def _layer_body(p_ref, h_ref, eps_ref, w1_ref, b1_ref, g1_ref, be1_ref,
                w2_ref, b2_ref, g2_ref, be2_ref, o_ref):
    e = eps_ref[0, 0]
    agg = p_ref[0, :N, :] + p_ref[1, :N, :] + (1.0 + e) * h_ref[:N, :]
    t = _matmul_t(agg, w1_ref[...]) + b1_ref[...]
    t = jnp.maximum(_bn(t, g1_ref[...], be1_ref[...]), 0.0)
    t = _matmul_t(t, w2_ref[...]) + b2_ref[...]
    _store_tab(o_ref, jnp.maximum(_bn(t, g2_ref[...], be2_ref[...]), 0.0))


@jax.jit
def _tc_layer(parts, h_tab, eps_l, w1, b1, g1, be1, w2, b2, g2, be2):
    r1 = lambda v: v.reshape(1, H)
    return pl.pallas_call(
        _layer_body,
        out_shape=jax.ShapeDtypeStruct((N + NZ, H), jnp.float32),
    )(parts, h_tab, eps_l.reshape(1, 1), w1, r1(b1), r1(g1), r1(be1),
      w2, r1(b2), r1(g2), r1(be2))


def _pool(h, seg):
    ids = lax.broadcasted_iota(jnp.int32, (M, 1), 0)
    onehot = (ids == seg).astype(jnp.float32)
    sums = lax.dot_general(onehot, h, (((1,), (0,)), ((), ())),
                           preferred_element_type=jnp.float32)
    counts = jnp.sum(onehot, axis=1, keepdims=True)
    return sums / jnp.maximum(counts, 1.0)


def _last_body(p_ref, h_ref, eps_ref, w1_ref, b1_ref, g1_ref, be1_ref,
               w2_ref, b2_ref, g2_ref, be2_ref, seg_ref, o_ref):
    e = eps_ref[0, 0]
    agg = p_ref[0, :N, :] + p_ref[1, :N, :] + (1.0 + e) * h_ref[:N, :]
    t = _matmul_t(agg, w1_ref[...]) + b1_ref[...]
    t = jnp.maximum(_bn(t, g1_ref[...], be1_ref[...]), 0.0)
    t = _matmul_t(t, w2_ref[...]) + b2_ref[...]
    t = jnp.maximum(_bn(t, g2_ref[...], be2_ref[...]), 0.0)
    o_ref[...] = _pool(t, seg_ref[...])


@jax.jit
def _tc_last(parts, h_tab, eps_l, w1, b1, g1, be1, w2, b2, g2, be2, seg):
    r1 = lambda v: v.reshape(1, H)
    return pl.pallas_call(
        _last_body,
        out_shape=jax.ShapeDtypeStruct((M, H), jnp.float32),
    )(parts, h_tab, eps_l.reshape(1, 1), w1, r1(b1), r1(g1), r1(be1),
      w2, r1(b2), r1(g2), r1(be2), seg.reshape(1, N))


def kernel(x, edge_index, segment_ids, W_in, b_in, eps,
           W1, b1, g1, beta1, W2, b2, g2, beta2):
    src = edge_index[0].astype(jnp.int32)
    dst = edge_index[1].astype(jnp.int32)
    pad = E_PAD - E
    pad_ar = jnp.arange(pad, dtype=jnp.int32)
    src_p = jnp.concatenate([src, N + (pad_ar % NZ)])
    dst_p = jnp.concatenate([dst, pad_ar % N])
    edges = jnp.stack([src_p.reshape(NW, TCH, C),
                       dst_p.reshape(NW, TCH, C)], axis=2)
    zeros = jnp.zeros((N_ACC, H), jnp.float32)

    h = _tc_proj(x, W_in, b_in)
    for l in range(DEPTH - 1):
        parts = _sc_agg(h, edges, zeros)
        h = _tc_layer(parts, h, eps[l], W1[l], b1[l], g1[l], beta1[l],
                      W2[l], b2[l], g2[l], beta2[l])
    l = DEPTH - 1
    parts = _sc_agg(h, edges, zeros)
    return _tc_last(parts, h, eps[l], W1[l], b1[l], g1[l], beta1[l],
                    W2[l], b2[l], g2[l], beta2[l],
                    segment_ids.astype(jnp.int32))


# speedup vs baseline: 1.0438x; 1.0438x over previous
import jax
import jax.numpy as jnp
from jax import lax
from jax.experimental import pallas as pl
from jax.experimental.pallas import tpu as pltpu
from jax.experimental.pallas import tpu_sc as plsc

N = 10000
E = 320000
H = 128
DEPTH = 3
M = 512
BN_EPS = 1e-5

NC = 2
NS = 16
NW = NC * NS
C = 72
TCH = 140
EW = C * TCH
E_PAD = EW * NW
NZ = 16
N_ACC = 10112
RPT = N_ACC // NS
NBUF = 5


def _sc_agg_body(h_hbm, e_hbm, zeros_hbm, out_hbm,
                 acc_sh, idx0, isem0, rows0, gsem0, ssem0,
                 idx1, isem1, rows1, gsem1, ssem1,
                 idx2, isem2, rows2, gsem2, ssem2,
                 idx3, isem3, rows3, gsem3, ssem3,
                 idx4, isem4, rows4, gsem4, ssem4):
    cid = lax.axis_index("c")
    sid = lax.axis_index("s")
    wid = sid * NC + cid

    pltpu.sync_copy(zeros_hbm.at[pl.ds(sid * RPT, RPT)],
                    acc_sh.at[pl.ds(sid * RPT, RPT)])
    plsc.subcore_barrier()

    idx = (idx0, idx1, idx2, idx3, idx4)
    isem = (isem0, isem1, isem2, isem3, isem4)
    rows = (rows0, rows1, rows2, rows3, rows4)
    gsem = (gsem0, gsem1, gsem2, gsem3, gsem4)
    ssem = (ssem0, ssem1, ssem2, ssem3, ssem4)

    def issue_idx(c, b):
        pltpu.async_copy(e_hbm.at[wid, c], idx[b], isem[b])

    def wait_idx(c, b):
        pltpu.make_async_copy(e_hbm.at[wid, c], idx[b], isem[b]).wait()

    def issue_gather(c, b):
        pltpu.async_copy(h_hbm.at[idx[b].at[0]], rows[b], gsem[b])

    def wait_gather(c, b):
        pltpu.make_async_copy(h_hbm.at[idx[b].at[0]], rows[b], gsem[b]).wait()

    def issue_scatter(c, b):
        pltpu.async_copy(rows[b], acc_sh.at[idx[b].at[1]], ssem[b], add=True)

    def wait_scatter(c, b):
        pltpu.make_async_copy(rows[b], acc_sh.at[idx[b].at[1]],
                              ssem[b]).wait()

    issue_idx(0, 0)
    issue_idx(1, 1)
    issue_idx(2, 2)
    wait_idx(0, 0)
    issue_gather(0, 0)
    wait_idx(1, 1)
    issue_gather(1, 1)
    for j in range(2):
        issue_idx(j + 3, j + 3)
        wait_idx(j + 2, j + 2)
        issue_gather(j + 2, j + 2)
        wait_gather(j, j)
        issue_scatter(j, j)

    @pl.loop(0, (TCH - 5) // NBUF)
    def _ring(k):
        for t in range(NBUF):
            j = 2 + k * NBUF + t
            b0 = (2 + t) % NBUF
            b2 = (4 + t) % NBUF
            b3 = t
            wait_scatter(j - 2, b3)
            issue_idx(j + 3, b3)
            wait_idx(j + 2, b2)
            issue_gather(j + 2, b2)
            wait_gather(j, b0)
            issue_scatter(j, b0)

    wait_idx(TCH - 1, (TCH - 1) % NBUF)
    issue_gather(TCH - 1, (TCH - 1) % NBUF)
    for j in range(TCH - 3, TCH):
        wait_gather(j, j % NBUF)
        issue_scatter(j, j % NBUF)
    for b in range(NBUF):
        wait_scatter(0, b)

    plsc.subcore_barrier()

    pltpu.sync_copy(acc_sh.at[pl.ds(sid * RPT, RPT)],
                    out_hbm.at[cid, pl.ds(sid * RPT, RPT)])


@jax.jit
def _sc_agg(h_tab, edges, zeros):
    mesh = plsc.VectorSubcoreMesh(core_axis_name="c", subcore_axis_name="s",
                                  num_cores=NC, num_subcores=NS)
    return pl.kernel(
        _sc_agg_body,
        out_type=jax.ShapeDtypeStruct((NC, N_ACC, H), jnp.float32),
        mesh=mesh,
        scratch_types=[
            pltpu.VMEM_SHARED((N_ACC, H), jnp.float32),
        ] + [
            s for _ in range(NBUF) for s in (
                pltpu.VMEM((2, C), jnp.int32),
                pltpu.SemaphoreType.DMA,
                pltpu.VMEM((C, H), jnp.float32),
                pltpu.SemaphoreType.DMA,
                pltpu.SemaphoreType.DMA,
            )
        ],
    )(h_tab, edges, zeros)


def _matmul_t(a, w):
    return lax.dot_general(a, w, (((1,), (1,)), ((), ())),
                           preferred_element_type=jnp.float32)


def _bn(t, g, b):
    mu = jnp.mean(t, axis=0, keepdims=True)
    d = t - mu
    var = jnp.mean(d * d, axis=0, keepdims=True)
    return g * (d * lax.rsqrt(var + BN_EPS)) + b


def _store_tab(o_ref, t):
    o_ref[:N, :] = t
    o_ref[N:, :] = jnp.zeros((NZ, H), jnp.float32)


def _proj_body(x_ref, w_ref, b_ref, o_ref):
    _store_tab(o_ref, jnp.maximum(
        _matmul_t(x_ref[...], w_ref[...]) + b_ref[...], 0.0))


@jax.jit
def _tc_proj(x, w_in, b_in):
    return pl.pallas_call(
        _proj_body,
        out_shape=jax.ShapeDtypeStruct((N + NZ, H), jnp.float32),
    )(x, w_in, b_in.reshape(1, H))


def _layer_body(p_ref, h_ref, eps_ref, w1_ref, b1_ref, g1_ref, be1_ref,
                w2_ref, b2_ref, g2_ref, be2_ref, o_ref):
    e = eps_ref[0, 0]
    agg = p_ref[0, :N, :] + p_ref[1, :N, :] + (1.0 + e) * h_ref[:N, :]
    t = _matmul_t(agg, w1_ref[...]) + b1_ref[...]
    t = jnp.maximum(_bn(t, g1_ref[...], be1_ref[...]), 0.0)
    t = _matmul_t(t, w2_ref[...]) + b2_ref[...]
    _store_tab(o_ref, jnp.maximum(_bn(t, g2_ref[...], be2_ref[...]), 0.0))


@jax.jit
def _tc_layer(parts, h_tab, eps_l, w1, b1, g1, be1, w2, b2, g2, be2):
    r1 = lambda v: v.reshape(1, H)
    return pl.pallas_call(
        _layer_body,
        out_shape=jax.ShapeDtypeStruct((N + NZ, H), jnp.float32),
    )(parts, h_tab, eps_l.reshape(1, 1), w1, r1(b1), r1(g1), r1(be1),
      w2, r1(b2), r1(g2), r1(be2))


def _pool(h, seg):
    ids = lax.broadcasted_iota(jnp.int32, (M, 1), 0)
    onehot = (ids == seg).astype(jnp.float32)
    sums = lax.dot_general(onehot, h, (((1,), (0,)), ((), ())),
                           preferred_element_type=jnp.float32)
    counts = jnp.sum(onehot, axis=1, keepdims=True)
    return sums / jnp.maximum(counts, 1.0)


def _last_body(p_ref, h_ref, eps_ref, w1_ref, b1_ref, g1_ref, be1_ref,
               w2_ref, b2_ref, g2_ref, be2_ref, seg_ref, o_ref):
    e = eps_ref[0, 0]
    agg = p_ref[0, :N, :] + p_ref[1, :N, :] + (1.0 + e) * h_ref[:N, :]
    t = _matmul_t(agg, w1_ref[...]) + b1_ref[...]
    t = jnp.maximum(_bn(t, g1_ref[...], be1_ref[...]), 0.0)
    t = _matmul_t(t, w2_ref[...]) + b2_ref[...]
    t = jnp.maximum(_bn(t, g2_ref[...], be2_ref[...]), 0.0)
    o_ref[...] = _pool(t, seg_ref[...])


@jax.jit
def _tc_last(parts, h_tab, eps_l, w1, b1, g1, be1, w2, b2, g2, be2, seg):
    r1 = lambda v: v.reshape(1, H)
    return pl.pallas_call(
        _last_body,
        out_shape=jax.ShapeDtypeStruct((M, H), jnp.float32),
    )(parts, h_tab, eps_l.reshape(1, 1), w1, r1(b1), r1(g1), r1(be1),
      w2, r1(b2), r1(g2), r1(be2), seg.reshape(1, N))


def kernel(x, edge_index, segment_ids, W_in, b_in, eps,
           W1, b1, g1, beta1, W2, b2, g2, beta2):
    src = edge_index[0].astype(jnp.int32)
    dst = edge_index[1].astype(jnp.int32)
    pad = E_PAD - E
    pad_ar = jnp.arange(pad, dtype=jnp.int32)
    src_p = jnp.concatenate([src, N + (pad_ar % NZ)])
    dst_p = jnp.concatenate([dst, pad_ar % N])
    edges = jnp.stack([src_p.reshape(NW, TCH, C),
                       dst_p.reshape(NW, TCH, C)], axis=2)
    zeros = jnp.zeros((N_ACC, H), jnp.float32)

    h = _tc_proj(x, W_in, b_in)
    for l in range(DEPTH - 1):
        parts = _sc_agg(h, edges, zeros)
        h = _tc_layer(parts, h, eps[l], W1[l], b1[l], g1[l], beta1[l],
                      W2[l], b2[l], g2[l], beta2[l])
    l = DEPTH - 1
    parts = _sc_agg(h, edges, zeros)
    return _tc_last(parts, h, eps[l], W1[l], b1[l], g1[l], beta1[l],
                    W2[l], b2[l], g2[l], beta2[l],
                    segment_ids.astype(jnp.int32))


# speedup vs baseline: 1.0591x; 1.0147x over previous
import jax
import jax.numpy as jnp
from jax import lax
from jax.experimental import pallas as pl
from jax.experimental.pallas import tpu as pltpu
from jax.experimental.pallas import tpu_sc as plsc

N = 10000
E = 320000
H = 128
DEPTH = 3
M = 512
BN_EPS = 1e-5

NC = 2
NS = 16
NW = NC * NS
C = 80
TCH = 125
EW = C * TCH
E_PAD = EW * NW
NZ = 16
N_ACC = 10112
RPT = N_ACC // NS
NRB = 4
NIB = 8


def _sc_agg_body(h_hbm, e_hbm, zeros_hbm, out_hbm, acc_sh,
                 idx0, isem0, idx1, isem1, idx2, isem2, idx3, isem3,
                 idx4, isem4, idx5, isem5, idx6, isem6, idx7, isem7,
                 rows0, gsem0, ssem0, rows1, gsem1, ssem1,
                 rows2, gsem2, ssem2, rows3, gsem3, ssem3):
    cid = lax.axis_index("c")
    sid = lax.axis_index("s")
    wid = sid * NC + cid

    pltpu.sync_copy(zeros_hbm.at[pl.ds(sid * RPT, RPT)],
                    acc_sh.at[pl.ds(sid * RPT, RPT)])
    plsc.subcore_barrier()

    idx = (idx0, idx1, idx2, idx3, idx4, idx5, idx6, idx7)
    isem = (isem0, isem1, isem2, isem3, isem4, isem5, isem6, isem7)
    rows = (rows0, rows1, rows2, rows3)
    gsem = (gsem0, gsem1, gsem2, gsem3)
    ssem = (ssem0, ssem1, ssem2, ssem3)

    def issue_idx(c, b):
        pltpu.async_copy(e_hbm.at[wid, c], idx[b], isem[b])

    def wait_idx(c, b):
        pltpu.make_async_copy(e_hbm.at[wid, c], idx[b], isem[b]).wait()

    def issue_gather(c, b, ib):
        pltpu.async_copy(h_hbm.at[idx[ib].at[0]], rows[b], gsem[b])

    def wait_gather(c, b, ib):
        pltpu.make_async_copy(h_hbm.at[idx[ib].at[0]], rows[b],
                              gsem[b]).wait()

    def issue_scatter(c, b, ib):
        pltpu.async_copy(rows[b], acc_sh.at[idx[ib].at[1]], ssem[b], add=True)

    def wait_scatter(c, b, ib):
        pltpu.make_async_copy(rows[b], acc_sh.at[idx[ib].at[1]],
                              ssem[b]).wait()

    issue_idx(0, 0)
    issue_idx(1, 1)
    issue_idx(2, 2)
    wait_idx(0, 0)
    issue_gather(0, 0, 0)
    wait_idx(1, 1)
    issue_gather(1, 1, 1)
    for j in range(2):
        issue_idx(j + 3, j + 3)
        wait_idx(j + 2, j + 2)
        issue_gather(j + 2, (j + 2) % NRB, j + 2)
        wait_gather(j, j, j)
        issue_scatter(j, j, j)

    @pl.loop(0, (TCH - 5) // NIB)
    def _ring(k):
        for t in range(NIB):
            j = 2 + k * NIB + t
            i3 = (5 + t) % NIB
            i2 = (4 + t) % NIB
            i0 = (2 + t) % NIB
            ir = t % NIB
            r2 = t % NRB
            r0 = (2 + t) % NRB
            issue_idx(j + 3, i3)
            wait_scatter(j - 2, r2, ir)
            wait_idx(j + 2, i2)
            issue_gather(j + 2, r2, i2)
            wait_gather(j, r0, i0)
            issue_scatter(j, r0, i0)

    wait_scatter(TCH - 5, (TCH - 5) % NRB, (TCH - 5) % NIB)
    wait_idx(TCH - 1, (TCH - 1) % NIB)
    issue_gather(TCH - 1, (TCH - 1) % NRB, (TCH - 1) % NIB)
    for j in range(TCH - 3, TCH):
        wait_gather(j, j % NRB, j % NIB)
        issue_scatter(j, j % NRB, j % NIB)
    for c in range(TCH - 4, TCH):
        if c == TCH - 5:
            continue
        wait_scatter(c, c % NRB, c % NIB)

    plsc.subcore_barrier()

    pltpu.sync_copy(acc_sh.at[pl.ds(sid * RPT, RPT)],
                    out_hbm.at[cid, pl.ds(sid * RPT, RPT)])


@jax.jit
def _sc_agg(h_tab, edges, zeros):
    mesh = plsc.VectorSubcoreMesh(core_axis_name="c", subcore_axis_name="s",
                                  num_cores=NC, num_subcores=NS)
    return pl.kernel(
        _sc_agg_body,
        out_type=jax.ShapeDtypeStruct((NC, N_ACC, H), jnp.float32),
        mesh=mesh,
        scratch_types=[
            pltpu.VMEM_SHARED((N_ACC, H), jnp.float32),
        ] + [
            s for _ in range(NIB) for s in (
                pltpu.VMEM((2, C), jnp.int32),
                pltpu.SemaphoreType.DMA,
            )
        ] + [
            s for _ in range(NRB) for s in (
                pltpu.VMEM((C, H), jnp.float32),
                pltpu.SemaphoreType.DMA,
                pltpu.SemaphoreType.DMA,
            )
        ],
    )(h_tab, edges, zeros)


def _matmul_t(a, w):
    return lax.dot_general(a, w, (((1,), (1,)), ((), ())),
                           preferred_element_type=jnp.float32)


def _bn(t, g, b):
    mu = jnp.mean(t, axis=0, keepdims=True)
    d = t - mu
    var = jnp.mean(d * d, axis=0, keepdims=True)
    return g * (d * lax.rsqrt(var + BN_EPS)) + b


def _store_tab(o_ref, t):
    o_ref[:N, :] = t
    o_ref[N:, :] = jnp.zeros((NZ, H), jnp.float32)


def _proj_body(x_ref, w_ref, b_ref, o_ref):
    _store_tab(o_ref, jnp.maximum(
        _matmul_t(x_ref[...], w_ref[...]) + b_ref[...], 0.0))


@jax.jit
def _tc_proj(x, w_in, b_in):
    return pl.pallas_call(
        _proj_body,
        out_shape=jax.ShapeDtypeStruct((N + NZ, H), jnp.float32),
    )(x, w_in, b_in.reshape(1, H))


def _layer_body(p_ref, h_ref, eps_ref, w1_ref, b1_ref, g1_ref, be1_ref,
                w2_ref, b2_ref, g2_ref, be2_ref, o_ref):
    e = eps_ref[0, 0]
    agg = p_ref[0, :N, :] + p_ref[1, :N, :] + (1.0 + e) * h_ref[:N, :]
    t = _matmul_t(agg, w1_ref[...]) + b1_ref[...]
    t = jnp.maximum(_bn(t, g1_ref[...], be1_ref[...]), 0.0)
    t = _matmul_t(t, w2_ref[...]) + b2_ref[...]
    _store_tab(o_ref, jnp.maximum(_bn(t, g2_ref[...], be2_ref[...]), 0.0))


@jax.jit
def _tc_layer(parts, h_tab, eps_l, w1, b1, g1, be1, w2, b2, g2, be2):
    r1 = lambda v: v.reshape(1, H)
    return pl.pallas_call(
        _layer_body,
        out_shape=jax.ShapeDtypeStruct((N + NZ, H), jnp.float32),
    )(parts, h_tab, eps_l.reshape(1, 1), w1, r1(b1), r1(g1), r1(be1),
      w2, r1(b2), r1(g2), r1(be2))


def _pool(h, seg):
    ids = lax.broadcasted_iota(jnp.int32, (M, 1), 0)
    onehot = (ids == seg).astype(jnp.float32)
    sums = lax.dot_general(onehot, h, (((1,), (0,)), ((), ())),
                           preferred_element_type=jnp.float32)
    counts = jnp.sum(onehot, axis=1, keepdims=True)
    return sums / jnp.maximum(counts, 1.0)


def _last_body(p_ref, h_ref, eps_ref, w1_ref, b1_ref, g1_ref, be1_ref,
               w2_ref, b2_ref, g2_ref, be2_ref, seg_ref, o_ref):
    e = eps_ref[0, 0]
    agg = p_ref[0, :N, :] + p_ref[1, :N, :] + (1.0 + e) * h_ref[:N, :]
    t = _matmul_t(agg, w1_ref[...]) + b1_ref[...]
    t = jnp.maximum(_bn(t, g1_ref[...], be1_ref[...]), 0.0)
    t = _matmul_t(t, w2_ref[...]) + b2_ref[...]
    t = jnp.maximum(_bn(t, g2_ref[...], be2_ref[...]), 0.0)
    o_ref[...] = _pool(t, seg_ref[...])


@jax.jit
def _tc_last(parts, h_tab, eps_l, w1, b1, g1, be1, w2, b2, g2, be2, seg):
    r1 = lambda v: v.reshape(1, H)
    return pl.pallas_call(
        _last_body,
        out_shape=jax.ShapeDtypeStruct((M, H), jnp.float32),
    )(parts, h_tab, eps_l.reshape(1, 1), w1, r1(b1), r1(g1), r1(be1),
      w2, r1(b2), r1(g2), r1(be2), seg.reshape(1, N))


def kernel(x, edge_index, segment_ids, W_in, b_in, eps,
           W1, b1, g1, beta1, W2, b2, g2, beta2):
    src = edge_index[0].astype(jnp.int32)
    dst = edge_index[1].astype(jnp.int32)
    pad = E_PAD - E
    pad_ar = jnp.arange(pad, dtype=jnp.int32)
    src_p = jnp.concatenate([src, N + (pad_ar % NZ)])
    dst_p = jnp.concatenate([dst, pad_ar % N])
    edges = jnp.stack([src_p.reshape(NW, TCH, C),
                       dst_p.reshape(NW, TCH, C)], axis=2)
    zeros = jnp.zeros((N_ACC, H), jnp.float32)

    h = _tc_proj(x, W_in, b_in)
    for l in range(DEPTH - 1):
        parts = _sc_agg(h, edges, zeros)
        h = _tc_layer(parts, h, eps[l], W1[l], b1[l], g1[l], beta1[l],
                      W2[l], b2[l], g2[l], beta2[l])
    l = DEPTH - 1
    parts = _sc_agg(h, edges, zeros)
    return _tc_last(parts, h, eps[l], W1[l], b1[l], g1[l], beta1[l],
                    W2[l], b2[l], g2[l], beta2[l],
                    segment_ids.astype(jnp.int32))
